# baseline (speedup 1.0000x reference)
import jax
import jax.numpy as jnp
from jax import lax
from jax.experimental import pallas as pl
from jax.experimental.pallas import tpu as pltpu
from jax.experimental.pallas import tpu_sc as plsc

EMBED = 32
OUT_D = 96
N_ROWS = 2 * 7 * 21
TAB_W = 512
BATCH = 16384
NC, NS = 2, 16
NW = NC * NS
NFG = 4
NBB = NW // NFG
FPW = OUT_D // NFG
BPW = BATCH // NBB
CHUNK = 128
NCH = BPW // CHUNK
L = 16


def _body(xg, xa, xo, tab, out, idx_v, fused_v, tab_v, out_v):
    c = lax.axis_index("c")
    s = lax.axis_index("s")
    wid = s * NC + c
    fg = wid % NFG
    bb = wid // NFG
    rbase = bb * NCH

    pltpu.sync_copy(tab.at[pl.ds(fg * FPW, FPW)], tab_v)
    pltpu.sync_copy(xg.at[pl.ds(rbase, NCH)], idx_v.at[0])
    pltpu.sync_copy(xa.at[pl.ds(rbase, NCH)], idx_v.at[1])
    pltpu.sync_copy(xo.at[pl.ds(rbase, NCH)], idx_v.at[2])

    for j in range(NCH):
        for i in range(CHUNK // L):
            sl = pl.ds(i * L, L)
            g = idx_v[0, j, sl]
            a = idx_v[1, j, sl]
            o = idx_v[2, j, sl]
            fused_v[j, sl] = g * 147 + a * 21 + o

    f_idx = [jnp.full((L,), f, jnp.int32) for f in range(FPW)]

    def _lookup(j):
        for i in range(CHUNK // L):
            idx16 = fused_v[j, pl.ds(i * L, L)]
            for f in range(FPW):
                out_v[f, pl.ds(j * CHUNK + i * L, L)] = plsc.load_gather(
                    tab_v, [f_idx[f], idx16])

    pl.loop(0, NCH)(_lookup)

    pltpu.sync_copy(out_v, out.at[pl.ds(fg * FPW, FPW), pl.ds(bb * BPW, BPW)])


def kernel(x1, W_gender, W_age, W_occupation):
    xg = x1[:, 0].reshape(BATCH // CHUNK, CHUNK)
    xa = x1[:, 1].reshape(BATCH // CHUNK, CHUNK)
    xo = x1[:, 2].reshape(BATCH // CHUNK, CHUNK)
    wg = jnp.pad(W_gender, ((0, 0), (0, OUT_D - EMBED)))
    wa = jnp.pad(W_age, ((0, 0), (EMBED, OUT_D - 2 * EMBED)))
    wo = jnp.pad(W_occupation, ((0, 0), (2 * EMBED, 0)))
    wcat = (
        wg[:, None, None, :] + wa[None, :, None, :] + wo[None, None, :, :]
    ).reshape(N_ROWS, OUT_D)
    tab = jnp.pad(wcat.T, ((0, 0), (0, TAB_W - N_ROWS)))
    k = pl.kernel(
        _body,
        out_type=jax.ShapeDtypeStruct((OUT_D, BATCH), jnp.float32),
        mesh=plsc.VectorSubcoreMesh(core_axis_name="c", subcore_axis_name="s"),
        scratch_types=[
            pltpu.VMEM((3, NCH, CHUNK), jnp.int32),
            pltpu.VMEM((NCH, CHUNK), jnp.int32),
            pltpu.VMEM((FPW, TAB_W), jnp.float32),
            pltpu.VMEM((FPW, BPW), jnp.float32),
        ],
        compiler_params=pltpu.CompilerParams(needs_layout_passes=False),
    )
    return k(xg, xa, xo, tab).T


# speedup vs baseline: 1.1404x; 1.1404x over previous
import jax
import jax.numpy as jnp
from jax import lax
from jax.experimental import pallas as pl
from jax.experimental.pallas import tpu as pltpu
from jax.experimental.pallas import tpu_sc as plsc

EMBED = 32
OUT_D = 96
PAD_D = 128
N_ROWS = 2 * 7 * 21
BATCH = 16384
NC, NS = 2, 16
NW = NC * NS
BPW = BATCH // NW
CHUNK = 128
NCH = BPW // CHUNK
L = 16


def _body(xs, wcat, out, idx_v, fused_v, tab_v, rows_v, rows96_v, sem, tsem, wsem):
    c = lax.axis_index("c")
    s = lax.axis_index("s")
    wid = s * NC + c
    rbase = wid * NCH
    base = wid * BPW

    @pl.when(s == 0)
    def _stage_table():
        pltpu.async_copy(wcat, tab_v, tsem)

    pltpu.sync_copy(xs.at[:, pl.ds(rbase, NCH)], idx_v)

    for j in range(NCH):
        for i in range(CHUNK // L):
            sl = pl.ds(i * L, L)
            g = idx_v[0, j, sl]
            a = idx_v[1, j, sl]
            o = idx_v[2, j, sl]
            fused_v[j, sl] = g * 147 + a * 21 + o

    @pl.when(s == 0)
    def _wait_table():
        pltpu.make_async_copy(wcat, tab_v, tsem).wait()

    plsc.subcore_barrier()

    gathers = [None, None]
    gathers[0] = pltpu.async_copy(tab_v.at[fused_v.at[0]], rows_v.at[0], sem)
    wbs = []
    for j in range(NCH):
        if j + 1 < NCH:
            gathers[(j + 1) % 2] = pltpu.async_copy(
                tab_v.at[fused_v.at[j + 1]], rows_v.at[(j + 1) % 2], sem)
        gathers[j % 2].wait()

        def _compact(i, jj=j):
            for cc in range(OUT_D // L):
                sl = pl.ds(cc * L, L)
                rows96_v[jj * CHUNK + i, sl] = rows_v[jj % 2, i, sl]

        pl.loop(0, CHUNK)(_compact)
        wbs.append(pltpu.async_copy(
            rows96_v.at[pl.ds(j * CHUNK, CHUNK)],
            out.at[pl.ds(base + j * CHUNK, CHUNK)], wsem))
    for d in wbs:
        d.wait()


def kernel(x1, W_gender, W_age, W_occupation):
    xs = x1.T.reshape(3, BATCH // CHUNK, CHUNK)
    wg = jnp.pad(W_gender, ((0, 0), (0, PAD_D - EMBED)))
    wa = jnp.pad(W_age, ((0, 0), (EMBED, PAD_D - 2 * EMBED)))
    wo = jnp.pad(W_occupation, ((0, 0), (2 * EMBED, PAD_D - 3 * EMBED)))
    wcat = (
        wg[:, None, None, :] + wa[None, :, None, :] + wo[None, None, :, :]
    ).reshape(N_ROWS, PAD_D)
    k = pl.kernel(
        _body,
        out_type=jax.ShapeDtypeStruct((BATCH, OUT_D), jnp.float32),
        mesh=plsc.VectorSubcoreMesh(core_axis_name="c", subcore_axis_name="s"),
        scratch_types=[
            pltpu.VMEM((3, NCH, CHUNK), jnp.int32),
            pltpu.VMEM((NCH, CHUNK), jnp.int32),
            pltpu.VMEM_SHARED((N_ROWS, PAD_D), jnp.float32),
            pltpu.VMEM((2, CHUNK, PAD_D), jnp.float32),
            pltpu.VMEM((BPW, OUT_D), jnp.float32),
            pltpu.SemaphoreType.DMA,
            pltpu.SemaphoreType.DMA,
            pltpu.SemaphoreType.DMA,
        ],
    )
    return k(xs, wcat)
